# baseline (speedup 1.0000x reference)
import functools

import jax
import jax.numpy as jnp
from jax import lax
from jax.experimental import pallas as pl
from jax.experimental.pallas import tpu as pltpu
from jax.experimental.pallas import tpu_sc as plsc

N = 10000
NPAD = 10112
RPT = NPAD // 16
E = 320000
CHUNK = 128
KK = 160
NC, NS = 2, 16
EPAD = NS * KK * CHUNK
NBUF = 2
IG = 40
D_IN = 128
D_HID = 128
D_OUT = 32
DH_HID = D_HID // NC
DH_OUT = D_OUT // NC
TCB = 2528


def _sc_mesh():
  return plsc.VectorSubcoreMesh(
      core_axis_name="c", subcore_axis_name="s", num_cores=NC, num_subcores=NS)


_SC_PARAMS = pltpu.CompilerParams(use_tc_tiling_on_sc=False)


def _make_deg_kernel():
  @functools.partial(
      pl.kernel,
      out_type=jax.ShapeDtypeStruct((NC, NPAD, 16), jnp.float32),
      mesh=_sc_mesh(),
      compiler_params=_SC_PARAMS,
      scratch_types=[
          pltpu.VMEM((KK // 2, CHUNK), jnp.int32),
          pltpu.VMEM((CHUNK, 16), jnp.float32),
          pltpu.VMEM_SHARED((NPAD, 16), jnp.float32),
      ],
  )
  def deg_kernel(dst_hbm, ones_hbm, half_hbm, out_hbm, dst_v, ones_v, acc):
    c = lax.axis_index("c")
    s = lax.axis_index("s")
    wid = c * NS + s
    pltpu.sync_copy(dst_hbm.at[wid], dst_v)
    pltpu.sync_copy(ones_hbm, ones_v)
    pltpu.sync_copy(half_hbm, acc.at[pl.ds(s * RPT, RPT)])
    plsc.subcore_barrier()

    @pl.loop(0, KK // 2)
    def _(j):
      pltpu.sync_copy(ones_v, acc.at[dst_v.at[j]], add=True)

    plsc.subcore_barrier()
    pltpu.sync_copy(acc.at[pl.ds(s * RPT, RPT)],
                    out_hbm.at[c, pl.ds(s * RPT, RPT)])

  return deg_kernel


def _make_agg_kernel(dh, final=False):
  extra_scratch = []
  if final:
    extra_scratch = [
        pltpu.VMEM((RPT, 16), jnp.float32),
        pltpu.VMEM((RPT, dh), jnp.float32),
        pltpu.VMEM((16,), jnp.float32),
    ]

  @functools.partial(
      pl.kernel,
      out_type=jax.ShapeDtypeStruct((NC, NPAD, dh), jnp.float32),
      mesh=_sc_mesh(),
      compiler_params=_SC_PARAMS,
      scratch_types=[
          pltpu.VMEM((IG, CHUNK), jnp.int32),
          pltpu.VMEM((IG, CHUNK), jnp.int32),
          [pltpu.VMEM((CHUNK, dh), jnp.float32)] * NBUF,
          pltpu.VMEM_SHARED((NPAD, dh), jnp.float32),
          pltpu.VMEM_SHARED((NPAD, dh), jnp.float32),
          [pltpu.SemaphoreType.DMA] * NBUF,
          [pltpu.SemaphoreType.DMA] * NBUF,
      ] + extra_scratch,
  )
  def agg_kernel(g_hbm, src_hbm, dst_hbm, *rest):
    if final:
      dinv_hbm, bias_hbm, out_hbm = rest[0], rest[1], rest[2]
      rest = rest[3:]
    else:
      out_hbm = rest[0]
      rest = rest[1:]
    src_v, dst_v, rows, gbuf, acc, gsem, ssem = rest[:7]
    if final:
      dinv_v, obuf, bias_v = rest[7], rest[8], rest[9]
    c = lax.axis_index("c")
    s = lax.axis_index("s")
    rs = pl.ds(s * RPT, RPT)
    pltpu.sync_copy(g_hbm.at[c, rs], gbuf.at[rs])
    pltpu.sync_copy(g_hbm.at[c, rs], acc.at[rs])
    plsc.subcore_barrier()

    @pl.loop(0, KK // IG)
    def _(grp):
      pltpu.sync_copy(src_hbm.at[s, pl.ds(grp * IG, IG)], src_v)
      pltpu.sync_copy(dst_hbm.at[s, pl.ds(grp * IG, IG)], dst_v)
      for b in range(NBUF):
        pltpu.async_copy(gbuf.at[src_v.at[b]], rows[b], gsem[b])

      @pl.loop(NBUF, IG, step=NBUF)
      def _(j):
        for b in range(NBUF):
          pltpu.make_async_copy(gbuf.at[src_v.at[j - NBUF + b]], rows[b],
                                gsem[b]).wait()
          pltpu.sync_copy(rows[b], acc.at[dst_v.at[j - NBUF + b]], add=True)
          pltpu.async_copy(gbuf.at[src_v.at[j + b]], rows[b], gsem[b])

      for b in range(NBUF):
        pltpu.make_async_copy(gbuf.at[src_v.at[IG - NBUF + b]], rows[b],
                              gsem[b]).wait()
        pltpu.sync_copy(rows[b], acc.at[dst_v.at[IG - NBUF + b]], add=True)

    plsc.subcore_barrier()
    if final:
      pltpu.sync_copy(acc.at[rs], obuf)
      pltpu.sync_copy(dinv_hbm.at[rs], dinv_v)
      pltpu.sync_copy(bias_hbm.at[c], bias_v)
      bias = bias_v[...]

      @pl.loop(0, RPT)
      def _(r):
        obuf[r, :] = obuf[r, :] * dinv_v[r, :] + bias

      pltpu.sync_copy(obuf, out_hbm.at[c, rs])
    else:
      pltpu.sync_copy(acc.at[rs], out_hbm.at[c, rs])

  return agg_kernel


def _tcmm_body(x_ref, w_ref, h_ref):
  h_ref[...] = jnp.dot(x_ref[...], w_ref[...],
                       preferred_element_type=jnp.float32)


def _tc1_body(h_ref, dega_ref, g_ref):
  deg = dega_ref[0, :, 0] + dega_ref[1, :, 0]
  dinv = lax.rsqrt(deg)[:, None]
  res = dinv * h_ref[...]
  g_ref[0] = res[:, :DH_HID]
  g_ref[1] = res[:, DH_HID:]


def _tc2_body(acc_ref, dega_ref, b1_ref, w2_ref, g2_ref, dinv_ref):
  deg = dega_ref[0, :, 0] + dega_ref[1, :, 0]
  dinv = lax.rsqrt(deg)[:, None]
  agg = jnp.concatenate([acc_ref[0], acc_ref[1]], axis=1)
  h = jnp.maximum(dinv * agg + b1_ref[...], 0.0)
  res = dinv * jnp.dot(h, w2_ref[...], preferred_element_type=jnp.float32)
  g2_ref[0] = res[:, :DH_OUT]
  g2_ref[1] = res[:, DH_OUT:]
  dinv_ref[...] = jnp.broadcast_to(dinv, (TCB, 16))


def kernel(x, edge_index, W1, b1, W2, b2):
  pad = jnp.full((EPAD - E,), N, dtype=jnp.int32)
  src = jnp.concatenate([edge_index[0].astype(jnp.int32), pad]
                        ).reshape(NS, KK, CHUNK)
  dst = jnp.concatenate([edge_index[1].astype(jnp.int32), pad]
                        ).reshape(NS, KK, CHUNK)
  dst32 = dst.reshape(NC * NS, KK // 2, CHUNK)
  xp = jnp.pad(x, ((0, NPAD - N), (0, 0)))
  ones16 = jnp.ones((CHUNK, 16), jnp.float32)
  half16 = jnp.full((RPT, 16), 0.5, jnp.float32)

  dega = _make_deg_kernel()(dst32, ones16, half16)

  grid = NPAD // TCB
  h1 = pl.pallas_call(
      _tcmm_body,
      grid=(grid,),
      in_specs=[
          pl.BlockSpec((TCB, D_IN), lambda i: (i, 0)),
          pl.BlockSpec((D_IN, D_HID), lambda i: (0, 0)),
      ],
      out_specs=pl.BlockSpec((TCB, D_HID), lambda i: (i, 0)),
      out_shape=jax.ShapeDtypeStruct((NPAD, D_HID), jnp.float32),
  )(xp, W1)

  g1 = pl.pallas_call(
      _tc1_body,
      grid=(grid,),
      in_specs=[
          pl.BlockSpec((TCB, D_HID), lambda i: (i, 0)),
          pl.BlockSpec((NC, TCB, 16), lambda i: (0, i, 0)),
      ],
      out_specs=pl.BlockSpec((NC, TCB, DH_HID), lambda i: (0, i, 0)),
      out_shape=jax.ShapeDtypeStruct((NC, NPAD, DH_HID), jnp.float32),
  )(h1, dega)

  acc1 = _make_agg_kernel(DH_HID)(g1, src, dst)

  g2, dinv16 = pl.pallas_call(
      _tc2_body,
      grid=(grid,),
      in_specs=[
          pl.BlockSpec((NC, TCB, DH_HID), lambda i: (0, i, 0)),
          pl.BlockSpec((NC, TCB, 16), lambda i: (0, i, 0)),
          pl.BlockSpec((1, D_HID), lambda i: (0, 0)),
          pl.BlockSpec((D_HID, D_OUT), lambda i: (0, 0)),
      ],
      out_specs=[
          pl.BlockSpec((NC, TCB, DH_OUT), lambda i: (0, i, 0)),
          pl.BlockSpec((TCB, 16), lambda i: (i, 0)),
      ],
      out_shape=[
          jax.ShapeDtypeStruct((NC, NPAD, DH_OUT), jnp.float32),
          jax.ShapeDtypeStruct((NPAD, 16), jnp.float32),
      ],
  )(acc1, dega, b1.reshape(1, D_HID), W2)

  outp = _make_agg_kernel(DH_OUT, final=True)(
      g2, src, dst, dinv16, b2.reshape(NC, DH_OUT))

  return jnp.concatenate([outp[0], outp[1]], axis=1)[:N]


# speedup vs baseline: 1.0538x; 1.0538x over previous
import functools

import jax
import jax.numpy as jnp
from jax import lax
from jax.experimental import pallas as pl
from jax.experimental.pallas import tpu as pltpu
from jax.experimental.pallas import tpu_sc as plsc

N = 10000
NPAD = 10112
RPT = NPAD // 16
E = 320000
CHUNK = 128
KK = 160
NC, NS = 2, 16
EPAD = NS * KK * CHUNK
NBUF = 2
IG = 80
D_IN = 128
D_HID = 128
D_OUT = 32
DH_HID = D_HID // NC
DH_OUT = D_OUT // NC
TCB = 2528


def _sc_mesh():
  return plsc.VectorSubcoreMesh(
      core_axis_name="c", subcore_axis_name="s", num_cores=NC, num_subcores=NS)


_SC_PARAMS = pltpu.CompilerParams(use_tc_tiling_on_sc=False)


def _make_deg_kernel():
  @functools.partial(
      pl.kernel,
      out_type=jax.ShapeDtypeStruct((NC, NPAD, 16), jnp.float32),
      mesh=_sc_mesh(),
      compiler_params=_SC_PARAMS,
      scratch_types=[
          pltpu.VMEM((KK // 2, CHUNK), jnp.int32),
          pltpu.VMEM((CHUNK, 16), jnp.float32),
          pltpu.VMEM_SHARED((NPAD, 16), jnp.float32),
      ],
  )
  def deg_kernel(dst_hbm, ones_hbm, half_hbm, out_hbm, dst_v, ones_v, acc):
    c = lax.axis_index("c")
    s = lax.axis_index("s")
    wid = c * NS + s
    pltpu.sync_copy(dst_hbm.at[wid], dst_v)
    pltpu.sync_copy(ones_hbm, ones_v)
    pltpu.sync_copy(half_hbm, acc.at[pl.ds(s * RPT, RPT)])
    plsc.subcore_barrier()

    @pl.loop(0, KK // 2)
    def _(j):
      pltpu.sync_copy(ones_v, acc.at[dst_v.at[j]], add=True)

    plsc.subcore_barrier()
    pltpu.sync_copy(acc.at[pl.ds(s * RPT, RPT)],
                    out_hbm.at[c, pl.ds(s * RPT, RPT)])

  return deg_kernel


def _make_agg_kernel(dh):
  @functools.partial(
      pl.kernel,
      out_type=jax.ShapeDtypeStruct((NC, NPAD, dh), jnp.float32),
      mesh=_sc_mesh(),
      compiler_params=_SC_PARAMS,
      scratch_types=[
          pltpu.VMEM((IG, CHUNK), jnp.int32),
          pltpu.VMEM((IG, CHUNK), jnp.int32),
          [pltpu.VMEM((CHUNK, dh), jnp.float32)] * NBUF,
          pltpu.VMEM_SHARED((NPAD, dh), jnp.float32),
          pltpu.VMEM_SHARED((NPAD, dh), jnp.float32),
          [pltpu.SemaphoreType.DMA] * NBUF,
      ],
  )
  def agg_kernel(g_hbm, src_hbm, dst_hbm, out_hbm,
                 src_v, dst_v, rows, gbuf, acc, gsem):
    c = lax.axis_index("c")
    s = lax.axis_index("s")
    rs = pl.ds(s * RPT, RPT)
    pltpu.sync_copy(g_hbm.at[c, rs], gbuf.at[rs])
    pltpu.sync_copy(g_hbm.at[c, rs], acc.at[rs])
    plsc.subcore_barrier()

    @pl.loop(0, KK // IG)
    def _(grp):
      pltpu.sync_copy(src_hbm.at[s, pl.ds(grp * IG, IG)], src_v)
      pltpu.sync_copy(dst_hbm.at[s, pl.ds(grp * IG, IG)], dst_v)
      for b in range(NBUF):
        pltpu.async_copy(gbuf.at[src_v.at[b]], rows[b], gsem[b])

      @pl.loop(NBUF, IG, step=NBUF)
      def _(j):
        for b in range(NBUF):
          pltpu.make_async_copy(gbuf.at[src_v.at[j - NBUF + b]], rows[b],
                                gsem[b]).wait()
          pltpu.sync_copy(rows[b], acc.at[dst_v.at[j - NBUF + b]], add=True)
          pltpu.async_copy(gbuf.at[src_v.at[j + b]], rows[b], gsem[b])

      for b in range(NBUF):
        pltpu.make_async_copy(gbuf.at[src_v.at[IG - NBUF + b]], rows[b],
                              gsem[b]).wait()
        pltpu.sync_copy(rows[b], acc.at[dst_v.at[IG - NBUF + b]], add=True)

    plsc.subcore_barrier()
    pltpu.sync_copy(acc.at[rs], out_hbm.at[c, rs])

  return agg_kernel


def _tcmm_body(x_ref, w_ref, h_ref):
  h_ref[...] = jnp.dot(x_ref[...], w_ref[...],
                       preferred_element_type=jnp.float32)


def _tc1_body(h_ref, dega_ref, g_ref):
  deg = dega_ref[0, :, 0] + dega_ref[1, :, 0]
  dinv = lax.rsqrt(deg)[:, None]
  res = dinv * h_ref[...]
  g_ref[0] = res[:, :DH_HID]
  g_ref[1] = res[:, DH_HID:]


def _tc2_body(acc_ref, dega_ref, b1_ref, w2_ref, g2_ref):
  deg = dega_ref[0, :, 0] + dega_ref[1, :, 0]
  dinv = lax.rsqrt(deg)[:, None]
  agg = jnp.concatenate([acc_ref[0], acc_ref[1]], axis=1)
  h = jnp.maximum(dinv * agg + b1_ref[...], 0.0)
  res = dinv * jnp.dot(h, w2_ref[...], preferred_element_type=jnp.float32)
  g2_ref[0] = res[:, :DH_OUT]
  g2_ref[1] = res[:, DH_OUT:]


def _tc3_body(acc_ref, dega_ref, b2_ref, out_ref):
  deg = dega_ref[0, :, 0] + dega_ref[1, :, 0]
  dinv = lax.rsqrt(deg)[:, None]
  agg = jnp.concatenate([acc_ref[0], acc_ref[1]], axis=1)
  out_ref[...] = dinv * agg + b2_ref[...]


def kernel(x, edge_index, W1, b1, W2, b2):
  pad = jnp.full((EPAD - E,), N, dtype=jnp.int32)
  src = jnp.concatenate([edge_index[0].astype(jnp.int32), pad]
                        ).reshape(NS, KK, CHUNK)
  dst = jnp.concatenate([edge_index[1].astype(jnp.int32), pad]
                        ).reshape(NS, KK, CHUNK)
  dst32 = dst.reshape(NC * NS, KK // 2, CHUNK)
  xp = jnp.pad(x, ((0, NPAD - N), (0, 0)))
  ones16 = jnp.ones((CHUNK, 16), jnp.float32)
  half16 = jnp.full((RPT, 16), 0.5, jnp.float32)

  dega = _make_deg_kernel()(dst32, ones16, half16)

  grid = NPAD // TCB
  h1 = pl.pallas_call(
      _tcmm_body,
      grid=(grid,),
      in_specs=[
          pl.BlockSpec((TCB, D_IN), lambda i: (i, 0)),
          pl.BlockSpec((D_IN, D_HID), lambda i: (0, 0)),
      ],
      out_specs=pl.BlockSpec((TCB, D_HID), lambda i: (i, 0)),
      out_shape=jax.ShapeDtypeStruct((NPAD, D_HID), jnp.float32),
  )(xp, W1)

  g1 = pl.pallas_call(
      _tc1_body,
      grid=(grid,),
      in_specs=[
          pl.BlockSpec((TCB, D_HID), lambda i: (i, 0)),
          pl.BlockSpec((NC, TCB, 16), lambda i: (0, i, 0)),
      ],
      out_specs=pl.BlockSpec((NC, TCB, DH_HID), lambda i: (0, i, 0)),
      out_shape=jax.ShapeDtypeStruct((NC, NPAD, DH_HID), jnp.float32),
  )(h1, dega)

  acc1 = _make_agg_kernel(DH_HID)(g1, src, dst)

  g2 = pl.pallas_call(
      _tc2_body,
      grid=(grid,),
      in_specs=[
          pl.BlockSpec((NC, TCB, DH_HID), lambda i: (0, i, 0)),
          pl.BlockSpec((NC, TCB, 16), lambda i: (0, i, 0)),
          pl.BlockSpec((1, D_HID), lambda i: (0, 0)),
          pl.BlockSpec((D_HID, D_OUT), lambda i: (0, 0)),
      ],
      out_specs=pl.BlockSpec((NC, TCB, DH_OUT), lambda i: (0, i, 0)),
      out_shape=jax.ShapeDtypeStruct((NC, NPAD, DH_OUT), jnp.float32),
  )(acc1, dega, b1.reshape(1, D_HID), W2)

  acc2 = _make_agg_kernel(DH_OUT)(g2, src, dst)

  out = pl.pallas_call(
      _tc3_body,
      grid=(grid,),
      in_specs=[
          pl.BlockSpec((NC, TCB, DH_OUT), lambda i: (0, i, 0)),
          pl.BlockSpec((NC, TCB, 16), lambda i: (0, i, 0)),
          pl.BlockSpec((1, D_OUT), lambda i: (0, 0)),
      ],
      out_specs=pl.BlockSpec((TCB, D_OUT), lambda i: (i, 0)),
      out_shape=jax.ShapeDtypeStruct((NPAD, D_OUT), jnp.float32),
  )(acc2, dega, b2.reshape(1, D_OUT))

  return out[:N]
